# baseline (speedup 1.0000x reference)
import jax
import jax.numpy as jnp
from jax.experimental import pallas as pl

_C = 0.7213475204444817


def _prep_body(h_ref, w_ref, vs_ref, vr_ref, mx_ref, fs_ref, fd_ref, cs_ref):
    d = w_ref.shape[1]
    m = jnp.dot(h_ref[...], w_ref[...], preferred_element_type=jnp.float32)
    mx_ref[:, :d] = m.astype(jnp.bfloat16)
    mx_ref[:, d:] = jnp.ones_like(mx_ref[:, d:])
    fs = jnp.dot(m, vs_ref[...], preferred_element_type=jnp.float32) * 0.5
    fd = jnp.dot(m, vr_ref[...], preferred_element_type=jnp.float32) * 0.5
    fs_ref[...] = fs.astype(jnp.bfloat16)
    fd_ref[...] = fd.astype(jnp.bfloat16)
    cs_ref[...] = jnp.sum(m, axis=0, keepdims=True)


def _attn_common(fs_ref, fd_ref, a_ref, mx_ref, cs_ref, o_ref, mask_out_ref):
    n = a_ref.shape[1]
    d = mx_ref.shape[1] - 1
    if mask_out_ref is not None:
        mk = a_ref[...].astype(jnp.float32).astype(jnp.bfloat16)
        mask_out_ref[...] = mk
    else:
        mk = a_ref[...]
    xh = fs_ref[...] + fd_ref[...]
    th = jnp.tanh(xh)
    e = jnp.exp2(th * jnp.bfloat16(_C)) * mk
    nd = jnp.dot(e, mx_ref[...], preferred_element_type=jnp.float32)
    num = nd[:, :d]
    denom = nd[:, d:]
    mean = cs_ref[...] * (1.0 / n)
    out = jnp.where(denom > 0.0, num / denom, mean)
    nrm = jnp.sqrt(jnp.sum(out * out, axis=1, keepdims=True))
    o_ref[...] = out / (nrm + 1e-12)


def _attn_body_emit(fs_ref, fd_ref, a_ref, mx_ref, cs_ref, o_ref, mask_ref):
    _attn_common(fs_ref, fd_ref, a_ref, mx_ref, cs_ref, o_ref, mask_ref)


def _attn_body(fs_ref, fd_ref, a_ref, mx_ref, cs_ref, o_ref):
    _attn_common(fs_ref, fd_ref, a_ref, mx_ref, cs_ref, o_ref, None)


def _gat_layer(H, adj, W, vs, vr, emit_mask, block_rows):
    n, _ = H.shape
    d_out = W.shape[1]
    br = min(block_rows, n)
    mx, fs, fd, cs = pl.pallas_call(
        _prep_body,
        out_shape=[
            jax.ShapeDtypeStruct((n, d_out + 1), jnp.bfloat16),
            jax.ShapeDtypeStruct((n, 1), jnp.bfloat16),
            jax.ShapeDtypeStruct((n, 1), jnp.bfloat16),
            jax.ShapeDtypeStruct((1, d_out), jnp.float32),
        ],
    )(H, W, vs, vr)
    fd_t = fd.reshape(1, n)
    grid = (n // br,)
    in_specs = [
        pl.BlockSpec((br, 1), lambda i: (i, 0)),
        pl.BlockSpec((1, n), lambda i: (0, 0)),
        pl.BlockSpec((br, n), lambda i: (i, 0)),
        pl.BlockSpec((n, d_out + 1), lambda i: (0, 0)),
        pl.BlockSpec((1, d_out), lambda i: (0, 0)),
    ]
    out_spec = pl.BlockSpec((br, d_out), lambda i: (i, 0))
    if emit_mask:
        out, mask16 = pl.pallas_call(
            _attn_body_emit,
            grid=grid,
            in_specs=in_specs,
            out_specs=[out_spec, pl.BlockSpec((br, n), lambda i: (i, 0))],
            out_shape=[
                jax.ShapeDtypeStruct((n, d_out), jnp.float32),
                jax.ShapeDtypeStruct((n, n), jnp.bfloat16),
            ],
        )(fs, fd_t, adj, mx, cs)
        return out, mask16
    out = pl.pallas_call(
        _attn_body,
        grid=grid,
        in_specs=in_specs,
        out_specs=out_spec,
        out_shape=jax.ShapeDtypeStruct((n, d_out), jnp.float32),
    )(fs, fd_t, adj, mx, cs)
    return out, None


def kernel(X, A, W0, vs0, vr0, W1, vs1, vr1, W2, vs2, vr2):
    H, mask16 = _gat_layer(X, A, W0, vs0, vr0, True, 512)
    H, _ = _gat_layer(H, mask16, W1, vs1, vr1, False, 1024)
    H, _ = _gat_layer(H, mask16, W2, vs2, vr2, False, 1024)
    return H


# speedup vs baseline: 1.0810x; 1.0810x over previous
import jax
import jax.numpy as jnp
from jax.experimental import pallas as pl

_C = 0.7213475204444817


def _prep_body(h_ref, w_ref, vs_ref, vr_ref, mx_ref, fs_ref, fd_ref, cs_ref):
    d = w_ref.shape[1]
    m = jnp.dot(h_ref[...], w_ref[...], preferred_element_type=jnp.float32)
    mx_ref[:, :d] = m.astype(jnp.bfloat16)
    mx_ref[:, d:] = jnp.ones_like(mx_ref[:, d:])
    fs = jnp.dot(m, vs_ref[...], preferred_element_type=jnp.float32) * 0.5
    fd = jnp.dot(m, vr_ref[...], preferred_element_type=jnp.float32) * 0.5
    fs_ref[...] = fs.astype(jnp.bfloat16)
    fd_ref[...] = fd.astype(jnp.bfloat16)
    cs_ref[...] = jnp.sum(m, axis=0, keepdims=True)


def _attn_common(fs_ref, fd_ref, a_ref, mx_ref, cs_ref, o_ref, mask_out_ref):
    n = a_ref.shape[1]
    d = mx_ref.shape[1] - 1
    if mask_out_ref is not None:
        a = a_ref[...]
        mask_out_ref[...] = a.astype(jnp.int8)
        mk = a.astype(jnp.float32).astype(jnp.bfloat16)
    else:
        mk = a_ref[...].astype(jnp.int32).astype(jnp.float32).astype(jnp.bfloat16)
    xh = fs_ref[...] + fd_ref[...]
    th = jnp.tanh(xh)
    e = jnp.exp2(th * jnp.bfloat16(_C)) * mk
    nd = jnp.dot(e, mx_ref[...], preferred_element_type=jnp.float32)
    num = nd[:, :d]
    denom = nd[:, d:]
    mean = cs_ref[...] * (1.0 / n)
    out = jnp.where(denom > 0.0, num / denom, mean)
    nrm = jnp.sqrt(jnp.sum(out * out, axis=1, keepdims=True))
    o_ref[...] = out / (nrm + 1e-12)


def _attn_body_emit(fs_ref, fd_ref, a_ref, mx_ref, cs_ref, o_ref, mask_ref):
    _attn_common(fs_ref, fd_ref, a_ref, mx_ref, cs_ref, o_ref, mask_ref)


def _attn_body(fs_ref, fd_ref, a_ref, mx_ref, cs_ref, o_ref):
    _attn_common(fs_ref, fd_ref, a_ref, mx_ref, cs_ref, o_ref, None)


def _gat_layer(H, adj, W, vs, vr, emit_mask, block_rows):
    n, _ = H.shape
    d_out = W.shape[1]
    br = min(block_rows, n)
    mx, fs, fd, cs = pl.pallas_call(
        _prep_body,
        out_shape=[
            jax.ShapeDtypeStruct((n, d_out + 1), jnp.bfloat16),
            jax.ShapeDtypeStruct((n, 1), jnp.bfloat16),
            jax.ShapeDtypeStruct((n, 1), jnp.bfloat16),
            jax.ShapeDtypeStruct((1, d_out), jnp.float32),
        ],
    )(H, W, vs, vr)
    fd_t = fd.reshape(1, n)
    grid = (n // br,)
    in_specs = [
        pl.BlockSpec((br, 1), lambda i: (i, 0)),
        pl.BlockSpec((1, n), lambda i: (0, 0)),
        pl.BlockSpec((br, n), lambda i: (i, 0)),
        pl.BlockSpec((n, d_out + 1), lambda i: (0, 0)),
        pl.BlockSpec((1, d_out), lambda i: (0, 0)),
    ]
    out_spec = pl.BlockSpec((br, d_out), lambda i: (i, 0))
    if emit_mask:
        out, mask16 = pl.pallas_call(
            _attn_body_emit,
            grid=grid,
            in_specs=in_specs,
            out_specs=[out_spec, pl.BlockSpec((br, n), lambda i: (i, 0))],
            out_shape=[
                jax.ShapeDtypeStruct((n, d_out), jnp.float32),
                jax.ShapeDtypeStruct((n, n), jnp.int8),
            ],
        )(fs, fd_t, adj, mx, cs)
        return out, mask16
    out = pl.pallas_call(
        _attn_body,
        grid=grid,
        in_specs=in_specs,
        out_specs=out_spec,
        out_shape=jax.ShapeDtypeStruct((n, d_out), jnp.float32),
    )(fs, fd_t, adj, mx, cs)
    return out, None


def kernel(X, A, W0, vs0, vr0, W1, vs1, vr1, W2, vs2, vr2):
    H, mask16 = _gat_layer(X, A, W0, vs0, vr0, True, 512)
    H, _ = _gat_layer(H, mask16, W1, vs1, vr1, False, 1024)
    H, _ = _gat_layer(H, mask16, W2, vs2, vr2, False, 1024)
    return H


# speedup vs baseline: 1.1713x; 1.0835x over previous
import jax
import jax.numpy as jnp
from jax.experimental import pallas as pl
from jax.experimental.pallas import tpu as pltpu

_C = 0.7213475204444817


def _layer_body(h_ref, w_ref, vs_ref, vrt_ref, a_ref, o_ref, mask_out_ref,
                mx_scr, fs_scr, fd_scr, cs_scr):
    i = pl.program_id(0)
    n = a_ref.shape[1]
    br = a_ref.shape[0]
    d = w_ref.shape[1]

    @pl.when(i == 0)
    def _prep():
        m = jnp.dot(h_ref[...], w_ref[...], preferred_element_type=jnp.float32)
        mx_scr[:, :d] = m.astype(jnp.bfloat16)
        mx_scr[:, d:] = jnp.ones_like(mx_scr[:, d:])
        fs = jnp.dot(m, vs_ref[...], preferred_element_type=jnp.float32) * 0.5
        fd = jax.lax.dot_general(vrt_ref[...], m, (((1,), (1,)), ((), ())),
                                 preferred_element_type=jnp.float32) * 0.5
        fs_scr[...] = fs.astype(jnp.bfloat16)
        fd_scr[...] = fd.astype(jnp.bfloat16)
        cs_scr[...] = jnp.sum(m, axis=0, keepdims=True)

    @pl.when(i > 0)
    def _attn():
        r0 = (i - 1) * br
        if mask_out_ref is not None:
            a = a_ref[...]
            mask_out_ref[...] = a.astype(jnp.int8)
            mk = a.astype(jnp.float32).astype(jnp.bfloat16)
        else:
            mk = a_ref[...].astype(jnp.int32).astype(jnp.float32).astype(jnp.bfloat16)
        fs = fs_scr[pl.ds(r0, br), :]
        ck = min(2048, n)
        nd = None
        for k in range(n // ck):
            xh = fs + fd_scr[:, k * ck:(k + 1) * ck]
            th = jnp.tanh(xh)
            e = jnp.exp2(th * jnp.bfloat16(_C)) * mk[:, k * ck:(k + 1) * ck]
            pd = jnp.dot(e, mx_scr[k * ck:(k + 1) * ck, :],
                         preferred_element_type=jnp.float32)
            nd = pd if nd is None else nd + pd
        num = nd[:, :d]
        denom = nd[:, d:]
        mean = cs_scr[...] * (1.0 / n)
        out = jnp.where(denom > 0.0, num / denom, mean)
        s = jnp.sum(out * out, axis=1, keepdims=True)
        o_ref[...] = out * jax.lax.rsqrt(s + 1e-30)


def _layer_body_emit(h_ref, w_ref, vs_ref, vrt_ref, a_ref, o_ref, mask_ref,
                     mx_scr, fs_scr, fd_scr, cs_scr):
    _layer_body(h_ref, w_ref, vs_ref, vrt_ref, a_ref, o_ref, mask_ref,
                mx_scr, fs_scr, fd_scr, cs_scr)


def _layer_body_noemit(h_ref, w_ref, vs_ref, vrt_ref, a_ref, o_ref,
                       mx_scr, fs_scr, fd_scr, cs_scr):
    _layer_body(h_ref, w_ref, vs_ref, vrt_ref, a_ref, o_ref, None,
                mx_scr, fs_scr, fd_scr, cs_scr)


def _gat_layer(H, adj, W, vs, vr, emit_mask, block_rows):
    n, d_in = H.shape
    d_out = W.shape[1]
    br = min(block_rows, n)
    vrt = vr.reshape(1, d_out)
    grid = (1 + n // br,)
    zero = lambda i: (0, 0)
    rowblk = lambda i: (jnp.maximum(i - 1, 0), 0)
    in_specs = [
        pl.BlockSpec((n, d_in), zero),
        pl.BlockSpec((d_in, d_out), zero),
        pl.BlockSpec((d_out, 1), zero),
        pl.BlockSpec((1, d_out), zero),
        pl.BlockSpec((br, n), rowblk),
    ]
    scratch_shapes = [
        pltpu.VMEM((n, d_out + 1), jnp.bfloat16),
        pltpu.VMEM((n, 1), jnp.bfloat16),
        pltpu.VMEM((1, n), jnp.bfloat16),
        pltpu.VMEM((1, d_out), jnp.float32),
    ]
    out_spec = pl.BlockSpec((br, d_out), rowblk)
    if emit_mask:
        out, mask8 = pl.pallas_call(
            _layer_body_emit,
            grid=grid,
            in_specs=in_specs,
            out_specs=[out_spec, pl.BlockSpec((br, n), rowblk)],
            out_shape=[
                jax.ShapeDtypeStruct((n, d_out), jnp.float32),
                jax.ShapeDtypeStruct((n, n), jnp.int8),
            ],
            scratch_shapes=scratch_shapes,
        )(H, W, vs, vrt, adj)
        return out, mask8
    out = pl.pallas_call(
        _layer_body_noemit,
        grid=grid,
        in_specs=in_specs,
        out_specs=out_spec,
        out_shape=jax.ShapeDtypeStruct((n, d_out), jnp.float32),
        scratch_shapes=scratch_shapes,
    )(H, W, vs, vrt, adj)
    return out, None


def kernel(X, A, W0, vs0, vr0, W1, vs1, vr1, W2, vs2, vr2):
    H, mask8 = _gat_layer(X, A, W0, vs0, vr0, True, 512)
    H, _ = _gat_layer(H, mask8, W1, vs1, vr1, False, 1024)
    H, _ = _gat_layer(H, mask8, W2, vs2, vr2, False, 1024)
    return H
